# Initial kernel scaffold; baseline (speedup 1.0000x reference)
import jax
import jax.numpy as jnp
from jax.experimental import pallas as pl


def kernel(x, edge_index, batch, W1, b1, W2, b2, W3, b3, W4, b4, W5, b5, fcW1, fcb1, fcW2, fcb2, fcW3, fcb3):
    raise NotImplementedError("write your pallas kernel here")


# speedup vs baseline: 4.4743x; 4.4743x over previous
import functools

import jax
import jax.numpy as jnp
from jax import lax
from jax.experimental import pallas as pl
from jax.experimental.pallas import tpu as pltpu
from jax.experimental.pallas import tpu_sc as plsc

N = 10000
NP = 10240
E = 320000
GQ = 128
NCORES = 2
NSUB = 16
TILES = NCORES * NSUB
KSTEP = 128
NSTEP = NP // KSTEP // 1 * 1
EPT = NP
NSTEP = EPT // KSTEP
EP = TILES * EPT
BR = 256
GRID = NP // BR
RPT = NP // NSUB


def _sc_scatter_fn(w):
    mesh = plsc.VectorSubcoreMesh(core_axis_name="c", subcore_axis_name="s", num_cores=NCORES, num_subcores=NSUB)

    @functools.partial(
        pl.kernel,
        out_type=jax.ShapeDtypeStruct((NCORES, NP, w), jnp.float32),
        mesh=mesh,
        scratch_types=[
            pltpu.VMEM((KSTEP,), jnp.int32),
            pltpu.VMEM((KSTEP,), jnp.int32),
            pltpu.VMEM((KSTEP, w), jnp.float32),
            pltpu.VMEM_SHARED((NP, w), jnp.float32),
            pltpu.SemaphoreType.DMA,
        ],
    )
    def body(t_hbm, src_hbm, dst_hbm, zeros_hbm, out_hbm, idx_s, idx_d, rows,
             acc, sem):
        cid = lax.axis_index("c")
        sid = lax.axis_index("s")
        tid = cid * NSUB + sid
        pltpu.sync_copy(zeros_hbm, acc.at[pl.ds(sid * RPT, RPT)])
        plsc.subcore_barrier()

        def step(k, carry):
            pltpu.sync_copy(src_hbm.at[tid, k], idx_s)
            pltpu.async_copy(t_hbm.at[idx_s], rows, sem).wait()
            pltpu.sync_copy(dst_hbm.at[tid, k], idx_d)
            pltpu.sync_copy(rows, acc.at[idx_d], add=True)
            return carry

        lax.fori_loop(0, NSTEP, step, 0)
        plsc.subcore_barrier()
        r0 = sid * RPT
        pltpu.sync_copy(acc.at[pl.ds(r0, RPT)], out_hbm.at[cid, pl.ds(r0, RPT)])

    return body


def _sc_deg_fn():
    w = 128
    mesh = plsc.VectorSubcoreMesh(core_axis_name="c", subcore_axis_name="s", num_cores=NCORES, num_subcores=NSUB)

    @functools.partial(
        pl.kernel,
        out_type=jax.ShapeDtypeStruct((NCORES, NP, w), jnp.float32),
        mesh=mesh,
        scratch_types=[
            pltpu.VMEM((KSTEP,), jnp.int32),
            pltpu.VMEM((KSTEP, w), jnp.float32),
            pltpu.VMEM_SHARED((NP, w), jnp.float32),
        ],
    )
    def body(ones_hbm, dst_hbm, zeros_hbm, out_hbm, idx_d, ones_v, acc):
        cid = lax.axis_index("c")
        sid = lax.axis_index("s")
        tid = cid * NSUB + sid
        pltpu.sync_copy(ones_hbm, ones_v)
        pltpu.sync_copy(zeros_hbm, acc.at[pl.ds(sid * RPT, RPT)])
        plsc.subcore_barrier()

        def step(k, carry):
            pltpu.sync_copy(dst_hbm.at[tid, k], idx_d)
            pltpu.sync_copy(ones_v, acc.at[idx_d], add=True)
            return carry

        lax.fori_loop(0, NSTEP, step, 0)
        plsc.subcore_barrier()
        r0 = sid * RPT
        pltpu.sync_copy(acc.at[pl.ds(r0, RPT)], out_hbm.at[cid, pl.ds(r0, RPT)])

    return body


_sc_scatter128 = _sc_scatter_fn(128)
_sc_deg = _sc_deg_fn()


def _row_spec(shape_tail):
    return pl.BlockSpec((BR,) + shape_tail, lambda i: (i,) + (0,) * len(shape_tail))


def _full_spec(shape):
    nd = len(shape)
    return pl.BlockSpec(shape, lambda i: (0,) * nd)


def _tc1(xp, W1, degp):
    def body(x_ref, w_ref, dp_ref, t_ref, u_ref):
        deg = dp_ref[0, :, 0:1] + dp_ref[1, :, 0:1] + 1.0
        u = lax.rsqrt(deg)
        u_ref[...] = u
        t_ref[...] = u * jnp.dot(x_ref[...], w_ref[...],
                                 preferred_element_type=jnp.float32)

    return pl.pallas_call(
        body,
        grid=(GRID,),
        in_specs=[_row_spec((128,)), _full_spec((128, 128)),
                  pl.BlockSpec((2, BR, 128), lambda i: (0, i, 0))],
        out_specs=[_row_spec((128,)), _row_spec((1,))],
        out_shape=[jax.ShapeDtypeStruct((NP, 128), jnp.float32),
                   jax.ShapeDtypeStruct((NP, 1), jnp.float32)],
    )(xp, W1, degp)


def _tc2(rawp, t1, u, b1):
    def body(r_ref, t_ref, u_ref, b_ref, o_ref):
        u = u_ref[...]
        h = jnp.maximum(u * (r_ref[0] + r_ref[1] + t_ref[...]) + b_ref[...], 0.0)
        o_ref[...] = u * h

    return pl.pallas_call(
        body,
        grid=(GRID,),
        in_specs=[pl.BlockSpec((2, BR, 128), lambda i: (0, i, 0)),
                  _row_spec((128,)), _row_spec((1,)), _full_spec((1, 128))],
        out_specs=_row_spec((128,)),
        out_shape=jax.ShapeDtypeStruct((NP, 128), jnp.float32),
    )(rawp, t1, u, b1)


def _tc3(rawp, t2, u, W2, b2):
    def body(r_ref, t_ref, u_ref, w_ref, b_ref, o_ref):
        u = u_ref[...]
        agg = u * (r_ref[0] + r_ref[1] + t_ref[...])
        y = jnp.dot(agg, w_ref[...], preferred_element_type=jnp.float32)
        o_ref[...] = u * jnp.maximum(y + b_ref[...], 0.0)

    return pl.pallas_call(
        body,
        grid=(GRID,),
        in_specs=[pl.BlockSpec((2, BR, 128), lambda i: (0, i, 0)),
                  _row_spec((128,)), _row_spec((1,)), _full_spec((128, 128)),
                  _full_spec((1, 128))],
        out_specs=_row_spec((128,)),
        out_shape=jax.ShapeDtypeStruct((NP, 128), jnp.float32),
    )(rawp, t2, u, W2, b2)


def _tc4(rawp, t3, u, W3, b3):
    def body(r_ref, t_ref, u_ref, w_ref, b_ref, o_ref):
        u = u_ref[...]
        agg = u * (r_ref[0] + r_ref[1] + t_ref[...])
        y = jnp.dot(agg, w_ref[...], preferred_element_type=jnp.float32)
        y = u * jnp.maximum(y + b_ref[...], 0.0)
        o_ref[0] = y[:, :128]
        o_ref[1] = y[:, 128:]

    return pl.pallas_call(
        body,
        grid=(GRID,),
        in_specs=[pl.BlockSpec((2, BR, 128), lambda i: (0, i, 0)),
                  _row_spec((128,)), _row_spec((1,)), _full_spec((128, 256)),
                  _full_spec((1, 256))],
        out_specs=pl.BlockSpec((2, BR, 128), lambda i: (0, i, 0)),
        out_shape=jax.ShapeDtypeStruct((2, NP, 128), jnp.float32),
    )(rawp, t3, u, W3, b3)


def _tc5(r4a, r4b, t4, u, W4, b4):
    def body(ra_ref, rb_ref, t_ref, u_ref, w_ref, b_ref, o_ref):
        u = u_ref[...]
        agg0 = u * (ra_ref[0] + ra_ref[1] + t_ref[0])
        agg1 = u * (rb_ref[0] + rb_ref[1] + t_ref[1])
        y = (jnp.dot(agg0, w_ref[:128, :], preferred_element_type=jnp.float32)
             + jnp.dot(agg1, w_ref[128:, :], preferred_element_type=jnp.float32))
        y = u * jnp.maximum(y + b_ref[...], 0.0)
        for s in range(4):
            o_ref[s] = y[:, s * 128:(s + 1) * 128]

    sl = pl.BlockSpec((2, BR, 128), lambda i: (0, i, 0))
    return pl.pallas_call(
        body,
        grid=(GRID,),
        in_specs=[sl, sl, sl, _row_spec((1,)), _full_spec((256, 512)),
                  _full_spec((1, 512))],
        out_specs=pl.BlockSpec((4, BR, 128), lambda i: (0, i, 0)),
        out_shape=jax.ShapeDtypeStruct((4, NP, 128), jnp.float32),
    )(r4a, r4b, t4, u, W4, b4)


def _tc6(r5a, r5b, r5c, r5d, t5, u, W5, b5):
    def body(ra, rb, rc, rd, t_ref, u_ref, w_ref, b_ref, o_ref):
        u = u_ref[...]
        rs = (ra, rb, rc, rd)
        y = b_ref[...]
        for s in range(4):
            agg = u * (rs[s][0] + rs[s][1] + t_ref[s])
            y = y + jnp.dot(agg, w_ref[s * 128:(s + 1) * 128, :],
                            preferred_element_type=jnp.float32)
        o_ref[...] = jnp.maximum(y, 0.0)

    sl = pl.BlockSpec((2, BR, 128), lambda i: (0, i, 0))
    return pl.pallas_call(
        body,
        grid=(GRID,),
        in_specs=[sl, sl, sl, sl,
                  pl.BlockSpec((4, BR, 128), lambda i: (0, i, 0)),
                  _row_spec((1,)), _full_spec((512, 1024)),
                  _full_spec((1, 1024))],
        out_specs=_row_spec((1024,)),
        out_shape=jax.ShapeDtypeStruct((NP, 1024), jnp.float32),
    )(r5a, r5b, r5c, r5d, t5, u, W5, b5)


CH = 128


def _tc_pool(batchp, h5):
    def body(b_ref, h_hbm, z_ref, scr, sem):
        g = pl.program_id(0)
        b = b_ref[...]
        start = jnp.sum((b < g).astype(jnp.int32))
        cnt = jnp.sum((b == g).astype(jnp.int32))
        start8 = pl.multiple_of((start // 8) * 8, 8)
        end = start + cnt
        ninf = jnp.float32(-jnp.inf)

        def cond(c):
            return start8 + c[0] < end

        def step(c):
            i, s, m = c
            off = pl.multiple_of(start8 + i, 8)
            cp = pltpu.make_async_copy(h_hbm.at[pl.ds(off, CH)], scr, sem)
            cp.start()
            cp.wait()
            gidx = lax.broadcasted_iota(jnp.int32, (CH, 1), 0) + (start8 + i)
            valid = (gidx >= start) & (gidx < end)
            chunk = scr[...]
            s = s + jnp.sum(jnp.where(valid, chunk, 0.0), axis=0, keepdims=True)
            m = jnp.maximum(m, jnp.max(jnp.where(valid, chunk, ninf), axis=0,
                                       keepdims=True))
            return i + CH, s, m

        z0 = jnp.zeros((1, 1024), jnp.float32)
        _, ssum, smax = lax.while_loop(cond, step, (0, z0, z0 + ninf))
        cntf = cnt.astype(jnp.float32)
        mean = ssum / jnp.maximum(cntf, 1.0)
        mx = jnp.where(cnt > 0, smax, 0.0)
        z_ref[...] = jnp.concatenate([mean, mx], axis=1)[None]

    return pl.pallas_call(
        body,
        grid=(GQ,),
        in_specs=[pl.BlockSpec((NP // 128, 128), lambda g: (0, 0)),
                  pl.BlockSpec(memory_space=pl.ANY)],
        out_specs=pl.BlockSpec((1, 1, 2048), lambda g: (g, 0, 0)),
        out_shape=jax.ShapeDtypeStruct((GQ, 1, 2048), jnp.float32),
        scratch_shapes=[pltpu.VMEM((CH, 1024), jnp.float32),
                        pltpu.SemaphoreType.DMA],
    )(batchp, h5)


def _tc_mlp(z, fcW1, fcb1, fcW2, fcb2, fcW3, fcb3):
    def body(z_ref, w1, b1, w2, b2, w3, b3, o_ref):
        a = jnp.maximum(jnp.dot(z_ref[...], w1[...],
                                preferred_element_type=jnp.float32) + b1[...], 0.0)
        a = jnp.maximum(jnp.dot(a, w2[...],
                                preferred_element_type=jnp.float32) + b2[...], 0.0)
        o_ref[...] = jnp.dot(a, w3[...],
                             preferred_element_type=jnp.float32) + b3[...]

    return pl.pallas_call(
        body,
        out_shape=jax.ShapeDtypeStruct((GQ, 7), jnp.float32),
    )(z, fcW1, fcb1, fcW2, fcb2, fcW3, fcb3)


def kernel(x, edge_index, batch, W1, b1, W2, b2, W3, b3, W4, b4, W5, b5,
           fcW1, fcb1, fcW2, fcb2, fcW3, fcb3):
    src, dst = edge_index[0], edge_index[1]
    xp = jnp.pad(x, ((0, NP - N), (0, 0)))
    src3 = jnp.pad(src, (0, EP - E)).reshape(TILES, NSTEP, KSTEP)
    dst3 = jnp.pad(dst, (0, EP - E),
                   constant_values=NP - 1).reshape(TILES, NSTEP, KSTEP)
    batchp = jnp.pad(batch, (0, NP - N),
                     constant_values=GQ).reshape(NP // 128, 128)
    z128 = jnp.zeros((RPT, 128), jnp.float32)
    ones128 = jnp.ones((KSTEP, 128), jnp.float32)
    W1p = jnp.pad(W1, ((0, 0), (0, 64)))
    W2p = jnp.pad(W2, ((0, 64), (0, 0)))
    b1p = jnp.pad(b1, (0, 64)).reshape(1, -1)

    degp = _sc_deg(ones128, dst3, z128)
    t1, u = _tc1(xp, W1p, degp)
    r1 = _sc_scatter128(t1, src3, dst3, z128)
    t2 = _tc2(r1, t1, u, b1p)
    r2 = _sc_scatter128(t2, src3, dst3, z128)
    t3 = _tc3(r2, t2, u, W2p, b2.reshape(1, -1))
    r3 = _sc_scatter128(t3, src3, dst3, z128)
    t4 = _tc4(r3, t3, u, W3, b3.reshape(1, -1))
    r4a = _sc_scatter128(t4[0], src3, dst3, z128)
    r4b = _sc_scatter128(t4[1], src3, dst3, z128)
    t5 = _tc5(r4a, r4b, t4, u, W4, b4.reshape(1, -1))
    r5 = [_sc_scatter128(t5[s], src3, dst3, z128) for s in range(4)]
    h5 = _tc6(r5[0], r5[1], r5[2], r5[3], t5, u, W5,
              b5.reshape(1, -1))
    z = _tc_pool(batchp, h5).reshape(GQ, 2048)
    return _tc_mlp(z, fcW1, fcb1.reshape(1, -1), fcW2, fcb2.reshape(1, -1),
                   fcW3, fcb3.reshape(1, -1))
